# baseline (speedup 1.0000x reference)
import functools

import jax
import jax.numpy as jnp
from jax import lax
from jax.experimental import pallas as pl
from jax.experimental.pallas import tpu as pltpu
from jax.experimental.pallas import tpu_sc as plsc

_R, _S = 65536, 128
_BR = 4096
_NC, _NS, _L = 2, 16, 16
_NW = _NC * _NS
_BPW = _R // _NW
_CH = 128
_NCH = _BPW // _CH


def _tc_index_body(w_ref, idx_ref):
    w = w_ref[...]
    i = lax.broadcasted_iota(jnp.int32, (_S, _S), 0)
    j = lax.broadcasted_iota(jnp.int32, (_S, _S), 1)
    tri = (i <= j).astype(jnp.float32)
    hi = w.astype(jnp.bfloat16).astype(jnp.float32)
    r1 = w - hi
    mid = r1.astype(jnp.bfloat16).astype(jnp.float32)
    lo = r1 - mid
    dn = (((0,), (1,)), ((), ()))
    cumT = (lax.dot_general(tri, lo, dn, preferred_element_type=jnp.float32)
            + lax.dot_general(tri, mid, dn, preferred_element_type=jnp.float32)
            + lax.dot_general(tri, hi, dn, preferred_element_type=jnp.float32))
    cnt = jnp.sum(jnp.where(cumT < 0.5, 1.0, 0.0), axis=0)
    cnt = jnp.minimum(cnt, float(_S - 1))
    row = lax.broadcasted_iota(jnp.int32, (_BR,), 0)
    base = pl.program_id(0) * _BR
    idx_ref[...] = (base + row) * _S + cnt.astype(jnp.int32)


_tc_index = pl.pallas_call(
    _tc_index_body,
    grid=(_R // _BR,),
    in_specs=[pl.BlockSpec((_BR, _S), lambda i: (i, 0))],
    out_specs=pl.BlockSpec((_BR,), lambda i: (i,)),
    out_shape=jax.ShapeDtypeStruct((_R,), jnp.int32),
)


def _sc_gather_body(idx_hbm, s_hbm, e_hbm, out_hbm, idx_v, sv, ev, sem_s, sem_e):
    wid = lax.axis_index("s") * _NC + lax.axis_index("c")
    base = wid * _BPW
    pltpu.sync_copy(idx_hbm.at[wid], idx_v)
    copies = []
    for j in range(_NCH):
        dst = pl.ds(j * _CH, _CH)
        copies.append(pltpu.async_copy(s_hbm.at[idx_v.at[j]], sv.at[dst], sem_s))
        copies.append(pltpu.async_copy(e_hbm.at[idx_v.at[j]], ev.at[dst], sem_e))
    for cp in copies:
        cp.wait()

    def body(i, carry):
        sl = pl.ds(i * _L, _L)
        sv[sl] = (sv[sl] + ev[sl]) * 0.5
        return carry

    lax.fori_loop(0, _BPW // _L, body, 0)
    pltpu.sync_copy(sv, out_hbm.at[pl.ds(base, _BPW)])


@functools.cache
def _make_sc_gather():
    mesh = plsc.VectorSubcoreMesh(core_axis_name="c", subcore_axis_name="s")
    return pl.kernel(
        _sc_gather_body,
        mesh=mesh,
        out_type=jax.ShapeDtypeStruct((_R,), jnp.float32),
        scratch_types=[
            pltpu.VMEM((_NCH, _CH), jnp.int32),
            pltpu.VMEM((_BPW,), jnp.float32),
            pltpu.VMEM((_BPW,), jnp.float32),
            pltpu.SemaphoreType.DMA,
            pltpu.SemaphoreType.DMA,
        ],
    )


def kernel(weights, starts, ends):
    w = weights.reshape(_R, _S)
    fidx = _tc_index(w).reshape(_NW, _NCH, _CH)
    s_flat = starts.reshape(_R * _S)
    e_flat = ends.reshape(_R * _S)
    depth = _make_sc_gather()(fidx, s_flat, e_flat)
    return depth.reshape(_R, 1)


# speedup vs baseline: 1.0709x; 1.0709x over previous
import functools

import jax
import jax.numpy as jnp
from jax import lax
from jax.experimental import pallas as pl
from jax.experimental.pallas import tpu as pltpu
from jax.experimental.pallas import tpu_sc as plsc

_R, _S = 65536, 128
_BR = 8192
_NC, _NS, _L = 2, 16, 16
_NW = _NC * _NS
_BPW = _R // _NW
_CH = 128
_NCH = _BPW // _CH


def _tc_index_body(w_ref, idx_ref):
    w = w_ref[...]
    i = lax.broadcasted_iota(jnp.int32, (_S, _S), 0)
    j = lax.broadcasted_iota(jnp.int32, (_S, _S), 1)
    tri = (i <= j).astype(jnp.float32)
    hi = w.astype(jnp.bfloat16).astype(jnp.float32)
    r1 = w - hi
    mid = r1.astype(jnp.bfloat16).astype(jnp.float32)
    lo = r1 - mid
    dn = (((0,), (1,)), ((), ()))
    cumT = (lax.dot_general(tri, lo, dn, preferred_element_type=jnp.float32)
            + lax.dot_general(tri, mid, dn, preferred_element_type=jnp.float32)
            + lax.dot_general(tri, hi, dn, preferred_element_type=jnp.float32))
    cnt = jnp.sum(jnp.where(cumT < 0.5, 1.0, 0.0), axis=0)
    cnt = jnp.minimum(cnt, float(_S - 1))
    row = lax.broadcasted_iota(jnp.int32, (_BR,), 0)
    base = pl.program_id(0) * _BR
    idx_ref[...] = (base + row) * _S + cnt.astype(jnp.int32)


_tc_index = pl.pallas_call(
    _tc_index_body,
    grid=(_R // _BR,),
    in_specs=[pl.BlockSpec((_BR, _S), lambda i: (i, 0))],
    out_specs=pl.BlockSpec((_BR,), lambda i: (i,)),
    out_shape=jax.ShapeDtypeStruct((_R,), jnp.int32),
)


def _sc_gather_body(idx_hbm, s_hbm, e_hbm, out_hbm, idx_v, sv, ev, sem_s, sem_e):
    wid = lax.axis_index("s") * _NC + lax.axis_index("c")
    base = wid * _BPW
    pltpu.sync_copy(idx_hbm.at[wid], idx_v)

    def fire(j, carry):
        dst = pl.ds(j * _CH, _CH)
        pltpu.make_async_copy(s_hbm.at[idx_v.at[j]], sv.at[dst], sem_s).start()
        pltpu.make_async_copy(e_hbm.at[idx_v.at[j]], ev.at[dst], sem_e).start()
        return carry

    def drain(j, carry):
        dst = pl.ds(j * _CH, _CH)
        pltpu.make_async_copy(s_hbm.at[idx_v.at[j]], sv.at[dst], sem_s).wait()
        pltpu.make_async_copy(e_hbm.at[idx_v.at[j]], ev.at[dst], sem_e).wait()
        return carry

    lax.fori_loop(0, _NCH, fire, 0)
    lax.fori_loop(0, _NCH, drain, 0)

    def body(i, carry):
        sl = pl.ds(i * _L, _L)
        sv[sl] = (sv[sl] + ev[sl]) * 0.5
        return carry

    lax.fori_loop(0, _BPW // _L, body, 0)
    pltpu.sync_copy(sv, out_hbm.at[pl.ds(base, _BPW)])


@functools.cache
def _make_sc_gather():
    mesh = plsc.VectorSubcoreMesh(core_axis_name="c", subcore_axis_name="s")
    return pl.kernel(
        _sc_gather_body,
        mesh=mesh,
        out_type=jax.ShapeDtypeStruct((_R,), jnp.float32),
        scratch_types=[
            pltpu.VMEM((_NCH, _CH), jnp.int32),
            pltpu.VMEM((_BPW,), jnp.float32),
            pltpu.VMEM((_BPW,), jnp.float32),
            pltpu.SemaphoreType.DMA,
            pltpu.SemaphoreType.DMA,
        ],
    )


def kernel(weights, starts, ends):
    w = weights.reshape(_R, _S)
    fidx = _tc_index(w).reshape(_NW, _NCH, _CH)
    s_flat = starts.reshape(_R * _S)
    e_flat = ends.reshape(_R * _S)
    depth = _make_sc_gather()(fidx, s_flat, e_flat)
    return depth.reshape(_R, 1)


# speedup vs baseline: 1.0790x; 1.0076x over previous
import functools

import jax
import jax.numpy as jnp
from jax import lax
from jax.experimental import pallas as pl
from jax.experimental.pallas import tpu as pltpu
from jax.experimental.pallas import tpu_sc as plsc

_R, _S = 65536, 128
_BR = 8192
_NC, _NS, _L = 2, 16, 16
_NW = _NC * _NS
_BPW = _R // _NW
_CH = 128
_NCH = _BPW // _CH


_HBR = _BR // 2


def _count(w):
    i = lax.broadcasted_iota(jnp.int32, (_S, _S), 0)
    j = lax.broadcasted_iota(jnp.int32, (_S, _S), 1)
    tri = (i <= j).astype(jnp.float32)
    hi = w.astype(jnp.bfloat16).astype(jnp.float32)
    r1 = w - hi
    mid = r1.astype(jnp.bfloat16).astype(jnp.float32)
    lo = r1 - mid
    dn = (((0,), (1,)), ((), ()))
    cumT = (lax.dot_general(tri, lo, dn, preferred_element_type=jnp.float32)
            + lax.dot_general(tri, mid, dn, preferred_element_type=jnp.float32)
            + lax.dot_general(tri, hi, dn, preferred_element_type=jnp.float32))
    return jnp.sum(jnp.where(cumT < 0.5, 1.0, 0.0), axis=0)


def _tc_index_body(wa_ref, wb_ref, idx_ref):
    cnt = jnp.concatenate([_count(wa_ref[...]), _count(wb_ref[...])])
    cnt = jnp.minimum(cnt, float(_S - 1))
    row = lax.broadcasted_iota(jnp.int32, (_BR,), 0)
    base = pl.program_id(0) * _BR
    idx_ref[...] = (base + row) * _S + cnt.astype(jnp.int32)


_tc_index = pl.pallas_call(
    _tc_index_body,
    grid=(_R // _BR,),
    in_specs=[pl.BlockSpec((_HBR, _S), lambda i: (2 * i, 0)),
              pl.BlockSpec((_HBR, _S), lambda i: (2 * i + 1, 0))],
    out_specs=pl.BlockSpec((_BR,), lambda i: (i,)),
    out_shape=jax.ShapeDtypeStruct((_R,), jnp.int32),
)


def _sc_gather_body(idx_hbm, s_hbm, e_hbm, out_hbm, idx_v, sv, ev, sem_s, sem_e):
    wid = lax.axis_index("s") * _NC + lax.axis_index("c")
    base = wid * _BPW
    pltpu.sync_copy(idx_hbm.at[wid], idx_v)

    def fire(j, carry):
        dst = pl.ds(j * _CH, _CH)
        pltpu.make_async_copy(s_hbm.at[idx_v.at[j]], sv.at[dst], sem_s).start()
        pltpu.make_async_copy(e_hbm.at[idx_v.at[j]], ev.at[dst], sem_e).start()
        return carry

    def drain(j, carry):
        dst = pl.ds(j * _CH, _CH)
        pltpu.make_async_copy(s_hbm.at[idx_v.at[j]], sv.at[dst], sem_s).wait()
        pltpu.make_async_copy(e_hbm.at[idx_v.at[j]], ev.at[dst], sem_e).wait()
        return carry

    lax.fori_loop(0, _NCH, fire, 0)
    lax.fori_loop(0, _NCH, drain, 0)

    def body(i, carry):
        sl = pl.ds(i * _L, _L)
        sv[sl] = (sv[sl] + ev[sl]) * 0.5
        return carry

    lax.fori_loop(0, _BPW // _L, body, 0)
    pltpu.sync_copy(sv, out_hbm.at[pl.ds(base, _BPW)])


@functools.cache
def _make_sc_gather():
    mesh = plsc.VectorSubcoreMesh(core_axis_name="c", subcore_axis_name="s")
    return pl.kernel(
        _sc_gather_body,
        mesh=mesh,
        out_type=jax.ShapeDtypeStruct((_R,), jnp.float32),
        scratch_types=[
            pltpu.VMEM((_NCH, _CH), jnp.int32),
            pltpu.VMEM((_BPW,), jnp.float32),
            pltpu.VMEM((_BPW,), jnp.float32),
            pltpu.SemaphoreType.DMA,
            pltpu.SemaphoreType.DMA,
        ],
    )


def kernel(weights, starts, ends):
    w = weights.reshape(_R, _S)
    fidx = _tc_index(w, w).reshape(_NW, _NCH, _CH)
    s_flat = starts.reshape(_R * _S)
    e_flat = ends.reshape(_R * _S)
    depth = _make_sc_gather()(fidx, s_flat, e_flat)
    return depth.reshape(_R, 1)


# speedup vs baseline: 1.0922x; 1.0122x over previous
import functools

import jax
import jax.numpy as jnp
from jax import lax
from jax.experimental import pallas as pl
from jax.experimental.pallas import tpu as pltpu
from jax.experimental.pallas import tpu_sc as plsc

_R, _S = 65536, 128
_BR = 8192
_NC, _NS, _L = 2, 16, 16
_NW = _NC * _NS
_BPW = _R // _NW
_CH = 128
_NCH = _BPW // _CH


_HBR = _BR // 2


def _count(w):
    i = lax.broadcasted_iota(jnp.int32, (_S, _S), 0)
    j = lax.broadcasted_iota(jnp.int32, (_S, _S), 1)
    tri = (i <= j).astype(jnp.float32)
    hi = w.astype(jnp.bfloat16).astype(jnp.float32)
    r1 = w - hi
    mid = r1.astype(jnp.bfloat16).astype(jnp.float32)
    lo = r1 - mid
    dn = (((0,), (1,)), ((), ()))
    cumT = (lax.dot_general(tri, lo, dn, preferred_element_type=jnp.float32)
            + lax.dot_general(tri, mid, dn, preferred_element_type=jnp.float32)
            + lax.dot_general(tri, hi, dn, preferred_element_type=jnp.float32))
    return jnp.sum(jnp.where(cumT < 0.5, 1.0, 0.0), axis=0)


def _tc_index_body(wa_ref, wb_ref, idx_ref):
    cnt = jnp.concatenate([_count(wa_ref[...]), _count(wb_ref[...])])
    cnt = jnp.minimum(cnt, float(_S - 1))
    row = lax.broadcasted_iota(jnp.int32, (_BR,), 0)
    base = pl.program_id(0) * _BR
    idx_ref[...] = (base + row) * _S + cnt.astype(jnp.int32)


_tc_index = pl.pallas_call(
    _tc_index_body,
    grid=(_R // _BR,),
    in_specs=[pl.BlockSpec((_HBR, _S), lambda i: (2 * i, 0)),
              pl.BlockSpec((_HBR, _S), lambda i: (2 * i + 1, 0))],
    out_specs=pl.BlockSpec((_BR,), lambda i: (i,)),
    out_shape=jax.ShapeDtypeStruct((_R,), jnp.int32),
)


def _sc_gather_body(idx_hbm, s_hbm, e_hbm, out_hbm, idx_v, sv, ev, sem_s, sem_e):
    wid = lax.axis_index("s") * _NC + lax.axis_index("c")
    base = wid * _BPW
    pltpu.sync_copy(idx_hbm.at[wid], idx_v)

    def fire(j, carry):
        dst = pl.ds(j * _CH, _CH)
        pltpu.make_async_copy(s_hbm.at[idx_v.at[j]], sv.at[dst], sem_s).start()
        pltpu.make_async_copy(e_hbm.at[idx_v.at[j]], ev.at[dst], sem_e).start()
        return carry

    def drain(j, carry):
        dst = pl.ds(j * _CH, _CH)
        pltpu.make_async_copy(s_hbm.at[idx_v.at[j]], sv.at[dst], sem_s).wait()
        pltpu.make_async_copy(e_hbm.at[idx_v.at[j]], ev.at[dst], sem_e).wait()
        return carry

    lax.fori_loop(0, _NCH, fire, 0)
    lax.fori_loop(0, _NCH, drain, 0)

    def body(i, carry):
        for u in range(4):
            sl = pl.ds((i * 4 + u) * _L, _L)
            sv[sl] = (sv[sl] + ev[sl]) * 0.5
        return carry

    lax.fori_loop(0, _BPW // (4 * _L), body, 0)
    pltpu.sync_copy(sv, out_hbm.at[pl.ds(base, _BPW)])


@functools.cache
def _make_sc_gather():
    mesh = plsc.VectorSubcoreMesh(core_axis_name="c", subcore_axis_name="s")
    return pl.kernel(
        _sc_gather_body,
        mesh=mesh,
        out_type=jax.ShapeDtypeStruct((_R,), jnp.float32),
        scratch_types=[
            pltpu.VMEM((_NCH, _CH), jnp.int32),
            pltpu.VMEM((_BPW,), jnp.float32),
            pltpu.VMEM((_BPW,), jnp.float32),
            pltpu.SemaphoreType.DMA,
            pltpu.SemaphoreType.DMA,
        ],
    )


def kernel(weights, starts, ends):
    w = weights.reshape(_R, _S)
    fidx = _tc_index(w, w).reshape(_NW, _NCH, _CH)
    s_flat = starts.reshape(_R * _S)
    e_flat = ends.reshape(_R * _S)
    depth = _make_sc_gather()(fidx, s_flat, e_flat)
    return depth.reshape(_R, 1)
